# baseline (speedup 1.0000x reference)
import jax
import jax.numpy as jnp
from jax.experimental import pallas as pl
from jax.experimental.pallas import tpu as pltpu

H = 128
W = 128
F_PAD = 1024
ROWS_PER_TILE = 16
P_TILE = ROWS_PER_TILE * W
N_TILES = H // ROWS_PER_TILE
BIG_IDX = 2 * F_PAD
_HIGHEST = jax.lax.Precision.HIGHEST


def _split3(v):
    hi = v.astype(jnp.bfloat16)
    r1 = v - hi.astype(jnp.float32)
    mid = r1.astype(jnp.bfloat16)
    lo = (r1 - mid.astype(jnp.float32)).astype(jnp.bfloat16)
    return hi, mid, lo


def _body(nv, nf, verts_ref, faces_ref, atab_ref, out_ref, ctab_ref):
    t = pl.program_id(0)

    @pl.when(t == 0)
    def _build_table():
        cidx = jax.lax.broadcasted_iota(jnp.int32, (F_PAD, 16), 1)
        ctab_ref[...] = jnp.where(cidx == 9, jnp.inf, 0.0)
        vids = jax.lax.broadcasted_iota(jnp.int32, (nf, nv), 1)
        fv = []
        for k in range(3):
            fk = faces_ref[:, k:k + 1]
            onehot = jnp.where(vids == fk, 1.0, 0.0)
            fv.append(jnp.dot(onehot, verts_ref[...], precision=_HIGHEST,
                              preferred_element_type=jnp.float32))
        x0, y0, z0 = fv[0][:, 0:1], fv[0][:, 1:2], fv[0][:, 2:3]
        x1, y1, z1 = fv[1][:, 0:1], fv[1][:, 1:2], fv[1][:, 2:3]
        x2, y2, z2 = fv[2][:, 0:1], fv[2][:, 1:2], fv[2][:, 2:3]
        dy12 = y1 - y2
        dx21 = x2 - x1
        dy20 = y2 - y0
        dx02 = x0 - x2
        denom = dy12 * dx02 + dx21 * (y0 - y2)
        valid = jnp.abs(denom) >= 1e-8
        zero = jnp.zeros_like(denom)
        ctab_ref[0:nf, 0:1] = jnp.where(valid, x2, zero)
        ctab_ref[0:nf, 1:2] = jnp.where(valid, y2, zero)
        ctab_ref[0:nf, 2:3] = jnp.where(valid, dy12, zero)
        ctab_ref[0:nf, 3:4] = jnp.where(valid, dx21, zero)
        ctab_ref[0:nf, 4:5] = jnp.where(valid, dy20, zero)
        ctab_ref[0:nf, 5:6] = jnp.where(valid, dx02, zero)
        ctab_ref[0:nf, 6:7] = jnp.where(valid, denom, 1.0)
        ctab_ref[0:nf, 7:8] = jnp.where(valid, z0, zero)
        ctab_ref[0:nf, 8:9] = jnp.where(valid, z1, zero)
        ctab_ref[0:nf, 9:10] = jnp.where(valid, z2, jnp.inf)

    x2 = ctab_ref[:, 0:1]
    y2 = ctab_ref[:, 1:2]
    dy12 = ctab_ref[:, 2:3]
    dx21 = ctab_ref[:, 3:4]
    dy20 = ctab_ref[:, 4:5]
    dx02 = ctab_ref[:, 5:6]
    denom_safe = ctab_ref[:, 6:7]
    z0 = ctab_ref[:, 7:8]
    z1 = ctab_ref[:, 8:9]
    z2 = ctab_ref[:, 9:10]

    j = jax.lax.broadcasted_iota(jnp.int32, (1, P_TILE), 1)
    row = (j // W) + ROWS_PER_TILE * t
    col = j % W
    py = -((2.0 * row.astype(jnp.float32) + 1.0) / H - 1.0)
    px = -((2.0 * col.astype(jnp.float32) + 1.0) / W - 1.0)

    dxp = px - x2
    dyp = py - y2
    w0 = (dy12 * dxp + dx21 * dyp) / denom_safe
    w1 = (dy20 * dxp + dx02 * dyp) / denom_safe
    w2 = 1.0 - w0 - w1
    inside = jnp.minimum(jnp.minimum(w0, w1), w2) >= 0.0
    z = w0 * z0 + w1 * z1 + w2 * z2
    zbuf = jnp.where(inside, z, jnp.inf)

    zmin = jnp.min(zbuf, axis=0, keepdims=True)
    hit = zmin < jnp.inf
    fidx = jax.lax.broadcasted_iota(jnp.int32, (F_PAD, P_TILE), 0)
    cand = jnp.where(zbuf == zmin, fidx, BIG_IDX)
    best = jnp.min(cand, axis=0, keepdims=True)
    onehot = fidx == best
    b0 = jnp.sum(jnp.where(onehot, w0, 0.0), axis=0, keepdims=True)
    b1 = jnp.sum(jnp.where(onehot, w1, 0.0), axis=0, keepdims=True)
    b2 = 1.0 - b0 - b1

    oh = jnp.where(onehot, 1.0, 0.0).astype(jnp.bfloat16)
    g = jnp.dot(atab_ref[...], oh,
                preferred_element_type=jnp.float32)
    ga = (g[0:9, :] + g[9:18, :]) + g[18:27, :]
    vals = b0 * ga[0:3, :] + b1 * ga[3:6, :] + b2 * ga[6:9, :]
    out_ref[0:3, :] = jnp.where(hit, vals, 0.0)
    out_ref[3:4, :] = jnp.where(hit, 1.0, 0.0)


def kernel(vertices, faces, h, w, attributes):
    N, nv, _ = vertices.shape
    nf = faces.shape[1]
    D = attributes.shape[-1]

    fixed = vertices * jnp.array([-1.0, -1.0, 1.0], dtype=vertices.dtype)
    hf = jnp.asarray(h, fixed.dtype)
    wf = jnp.asarray(w, fixed.dtype)
    one = jnp.asarray(1.0, fixed.dtype)
    sx = jnp.where(hf > wf, one, wf / hf)
    sy = jnp.where(hf > wf, hf / wf, one)
    fixed = (fixed * jnp.stack([sx, sy, one])).astype(jnp.float32)[0]

    faces32 = faces[0].astype(jnp.int32)

    attrs9 = jnp.zeros((3 * D, F_PAD), jnp.float32).at[:, 0:nf].set(
        jnp.transpose(attributes[0].reshape(nf, 3 * D)))
    ahi, amid, alo = _split3(attrs9)
    atab = jnp.concatenate(
        [ahi, amid, alo,
         jnp.zeros((32 - 9 * D, F_PAD), jnp.bfloat16)], axis=0)

    import functools
    out_cm = pl.pallas_call(
        functools.partial(_body, nv, nf),
        grid=(N_TILES,),
        in_specs=[
            pl.BlockSpec((nv, 3), lambda t: (0, 0)),
            pl.BlockSpec((nf, 3), lambda t: (0, 0)),
            pl.BlockSpec((32, F_PAD), lambda t: (0, 0)),
        ],
        out_specs=pl.BlockSpec((D + 1, P_TILE), lambda t: (0, t)),
        out_shape=jax.ShapeDtypeStruct((D + 1, H * W), jnp.float32),
        scratch_shapes=[pltpu.VMEM((F_PAD, 16), jnp.float32)],
    )(fixed, faces32, atab)

    return out_cm.reshape(N, D + 1, H, W)


# speedup vs baseline: 1.1192x; 1.1192x over previous
import jax
import jax.numpy as jnp
from jax.experimental import pallas as pl

H = 128
W = 128
F_PAD = 1024
V_PAD = 640
ROWS_PER_TILE = 8
P_TILE = ROWS_PER_TILE * W
N_TILES = H // ROWS_PER_TILE
BIG_IDX = 2 * F_PAD
_HIGHEST = jax.lax.Precision.HIGHEST


def _split3(v):
    hi = v.astype(jnp.bfloat16)
    r1 = v - hi.astype(jnp.float32)
    mid = r1.astype(jnp.bfloat16)
    lo = (r1 - mid.astype(jnp.float32)).astype(jnp.bfloat16)
    return hi, mid, lo


def _face_table_body(verts_ref, faces_ref, valids_ref, attrs_ref,
                     table_ref, atab_ref):
    vids = jax.lax.broadcasted_iota(jnp.int32, (V_PAD, F_PAD), 0)
    fv = []
    for k in range(3):
        fk = faces_ref[k:k + 1, :]
        onehot = jnp.where(vids == fk, 1.0, 0.0)
        fv.append(jnp.dot(verts_ref[...], onehot, precision=_HIGHEST,
                          preferred_element_type=jnp.float32))
    x0, y0, z0 = fv[0][0:1, :], fv[0][1:2, :], fv[0][2:3, :]
    x1, y1, z1 = fv[1][0:1, :], fv[1][1:2, :], fv[1][2:3, :]
    x2, y2, z2 = fv[2][0:1, :], fv[2][1:2, :], fv[2][2:3, :]
    dy12 = y1 - y2
    dx21 = x2 - x1
    dy20 = y2 - y0
    dx02 = x0 - x2
    denom = dy12 * dx02 + dx21 * (y0 - y2)
    valid = (jnp.abs(denom) >= 1e-8) & (valids_ref[0:1, :] > 0.5)
    denom_safe = jnp.where(valid, denom, 1.0)
    zero = jnp.zeros_like(denom)
    table_ref[0:1, :] = jnp.where(valid, x2, zero)
    table_ref[1:2, :] = jnp.where(valid, y2, zero)
    table_ref[2:3, :] = jnp.where(valid, dy12, zero)
    table_ref[3:4, :] = jnp.where(valid, dx21, zero)
    table_ref[4:5, :] = jnp.where(valid, dy20, zero)
    table_ref[5:6, :] = jnp.where(valid, dx02, zero)
    table_ref[6:7, :] = denom_safe
    table_ref[7:8, :] = jnp.where(valid, z0, zero)
    table_ref[8:9, :] = jnp.where(valid, z1, zero)
    table_ref[9:10, :] = jnp.where(valid, z2, jnp.inf)
    table_ref[10:16, :] = jnp.zeros((6, F_PAD), jnp.float32)

    ahi, amid, alo = _split3(attrs_ref[...])
    atab_ref[:, 0:9] = ahi
    atab_ref[:, 9:18] = amid
    atab_ref[:, 18:27] = alo
    atab_ref[:, 27:32] = jnp.zeros((F_PAD, 5), jnp.bfloat16)


def _raster_body(table_ref, atab_ref, out_ref):
    t = pl.program_id(0)
    x2 = table_ref[0:1, :]
    y2 = table_ref[1:2, :]
    dy12 = table_ref[2:3, :]
    dx21 = table_ref[3:4, :]
    dy20 = table_ref[4:5, :]
    dx02 = table_ref[5:6, :]
    denom_safe = table_ref[6:7, :]
    z0 = table_ref[7:8, :]
    z1 = table_ref[8:9, :]
    z2 = table_ref[9:10, :]

    p = jax.lax.broadcasted_iota(jnp.int32, (P_TILE, 1), 0)
    row = (p // W) + ROWS_PER_TILE * t
    col = p % W
    py = -((2.0 * row.astype(jnp.float32) + 1.0) / H - 1.0)
    px = -((2.0 * col.astype(jnp.float32) + 1.0) / W - 1.0)

    dxp = px - x2
    dyp = py - y2
    w0 = (dy12 * dxp + dx21 * dyp) / denom_safe
    w1 = (dy20 * dxp + dx02 * dyp) / denom_safe
    w2 = 1.0 - w0 - w1
    inside = jnp.minimum(jnp.minimum(w0, w1), w2) >= 0.0
    z = w0 * z0 + w1 * z1 + w2 * z2
    zbuf = jnp.where(inside, z, jnp.inf)

    zmin = jnp.min(zbuf, axis=1, keepdims=True)
    hit = zmin < jnp.inf
    fidx = jax.lax.broadcasted_iota(jnp.int32, (P_TILE, F_PAD), 1)
    cand = jnp.where(zbuf == zmin, fidx, BIG_IDX)
    best = jnp.min(cand, axis=1, keepdims=True)
    onehot = fidx == best
    b0 = jnp.sum(jnp.where(onehot, w0, 0.0), axis=1, keepdims=True)
    b1 = jnp.sum(jnp.where(onehot, w1, 0.0), axis=1, keepdims=True)
    b2 = 1.0 - b0 - b1

    oh = jnp.where(onehot, 1.0, 0.0).astype(jnp.bfloat16)
    g = jnp.dot(oh, atab_ref[...],
                preferred_element_type=jnp.float32)
    ga = (g[:, 0:9] + g[:, 9:18]) + g[:, 18:27]
    vals = b0 * ga[:, 0:3] + b1 * ga[:, 3:6] + b2 * ga[:, 6:9]
    out_ref[:, 0:3] = jnp.where(hit, vals, 0.0)
    out_ref[:, 3:4] = jnp.where(hit, 1.0, 0.0)


def kernel(vertices, faces, h, w, attributes):
    N, nv, _ = vertices.shape
    nf = faces.shape[1]
    D = attributes.shape[-1]

    fixed = vertices * jnp.array([-1.0, -1.0, 1.0], dtype=vertices.dtype)
    hf = jnp.asarray(h, fixed.dtype)
    wf = jnp.asarray(w, fixed.dtype)
    one = jnp.asarray(1.0, fixed.dtype)
    sx = jnp.where(hf > wf, one, wf / hf)
    sy = jnp.where(hf > wf, hf / wf, one)
    fixed = (fixed * jnp.stack([sx, sy, one])).astype(jnp.float32)

    verts_t = jnp.zeros((8, V_PAD), jnp.float32).at[0:3, 0:nv].set(
        jnp.transpose(fixed[0]))
    faces_t = jnp.zeros((8, F_PAD), jnp.int32).at[0:3, 0:nf].set(
        jnp.transpose(faces[0]).astype(jnp.int32))
    valids = jnp.zeros((8, F_PAD), jnp.float32).at[0, 0:nf].set(1.0)
    attrs_flat = jnp.zeros((F_PAD, 3 * D), jnp.float32).at[0:nf, :].set(
        attributes[0].reshape(nf, 3 * D))

    table, atab = pl.pallas_call(
        _face_table_body,
        out_shape=(jax.ShapeDtypeStruct((16, F_PAD), jnp.float32),
                   jax.ShapeDtypeStruct((F_PAD, 32), jnp.bfloat16)),
    )(verts_t, faces_t, valids, attrs_flat)

    out_flat = pl.pallas_call(
        _raster_body,
        grid=(N_TILES,),
        in_specs=[
            pl.BlockSpec((16, F_PAD), lambda t: (0, 0)),
            pl.BlockSpec((F_PAD, 32), lambda t: (0, 0)),
        ],
        out_specs=pl.BlockSpec((P_TILE, D + 1), lambda t: (t, 0)),
        out_shape=jax.ShapeDtypeStruct((H * W, D + 1), jnp.float32),
    )(table, atab)

    return jnp.transpose(out_flat).reshape(N, D + 1, H, W)
